# baseline (speedup 1.0000x reference)
import functools
import math

import jax
import jax.numpy as jnp
from jax import lax
from jax.experimental import pallas as pl
from jax.experimental.pallas import tpu as pltpu
from jax.experimental.pallas import tpu_sc as plsc

_BATCH = 4096
_WINDOW = 200
_EMBED = 64
_NC, _NS = 2, 16
_NW = _NC * _NS
_BSL = _BATCH // _NW
_SCALE = math.sqrt(_EMBED)
_L = 16
_ET = _EMBED // 8

_mesh = plsc.VectorSubcoreMesh(core_axis_name="c", subcore_axis_name="s")


@functools.partial(
    pl.kernel,
    out_type=jax.ShapeDtypeStruct((_WINDOW, _ET, _NW, 8, _BSL), jnp.float32),
    mesh=_mesh,
    scratch_types=[
        pltpu.VMEM((_WINDOW, _BSL), jnp.int32),
        pltpu.VMEM((_BSL, _EMBED), jnp.float32),
        pltpu.VMEM((_BSL, _EMBED), jnp.float32),
        pltpu.VMEM((_ET, 8, _BSL), jnp.float32),
        pltpu.VMEM((_ET, 8, _BSL), jnp.float32),
        pltpu.VMEM((_WINDOW, _EMBED), jnp.float32),
        pltpu.SemaphoreType.DMA,
        pltpu.SemaphoreType.DMA,
        pltpu.SemaphoreType.DMA,
        pltpu.SemaphoreType.DMA,
    ],
    compiler_params=pltpu.CompilerParams(
        use_tc_tiling_on_sc=False, needs_layout_passes=False),
)
def _emb_pe_kernel(xt_hbm, table_hbm, pos_hbm, out_hbm,
                   idx_v, gbuf0, gbuf1, obuf0, obuf1, pos_v,
                   sg0, sg1, so0, so1):
    wid = lax.axis_index("s") * _NC + lax.axis_index("c")
    b0 = wid * _BSL
    gbufs, obufs = (gbuf0, gbuf1), (obuf0, obuf1)
    sgs, sos = (sg0, sg1), (so0, so1)

    pltpu.sync_copy(pos_hbm, pos_v)
    pltpu.sync_copy(xt_hbm.at[:, pl.ds(b0, _BSL)], idx_v)

    iota = jnp.arange(_L, dtype=jnp.int32)
    i0s = [(s * _L + iota) >> 3 for s in range(_EMBED // _L)]
    i1s = [(s * _L + iota) & 7 for s in range(_EMBED // _L)]

    def fire_gather(w, b):
        return pltpu.async_copy(
            table_hbm.at[idx_v.at[w]], gbufs[b], sgs[b])

    def wait_gather(w, b):
        pltpu.make_async_copy(
            table_hbm.at[idx_v.at[w]], gbufs[b], sgs[b]).wait()

    def fire_out(w, b):
        for e in range(_ET):
            pltpu.async_copy(obufs[b].at[e], out_hbm.at[w, e, wid], sos[b])

    def drain_out(w, b):
        for e in range(_ET):
            pltpu.make_async_copy(
                obufs[b].at[e], out_hbm.at[w, e, wid], sos[b]).wait()

    def compute(w, b):
        gbuf, obuf = gbufs[b], obufs[b]
        ps = [pos_v[w, pl.ds(s * _L, _L)] for s in range(_EMBED // _L)]

        @pl.loop(0, _BSL, unroll=4)
        def _col(c):
            i2 = jnp.full((_L,), c, jnp.int32)
            for s in range(_EMBED // _L):
                v = gbuf[c, pl.ds(s * _L, _L)] * _SCALE + ps[s]
                plsc.store_scatter(obuf, (i0s[s], i1s[s], i2), v)

    fire_gather(0, 0)

    @pl.loop(0, _WINDOW, step=2)
    def _wloop(wb):
        for b in range(2):
            w = wb + b

            @pl.when(w + 1 < _WINDOW)
            def _():
                fire_gather(w + 1, 1 - b)

            wait_gather(w, b)

            @pl.when(w >= 2)
            def _():
                drain_out(w - 2, b)

            compute(w, b)
            fire_out(w, b)

    drain_out(_WINDOW - 2, 0)
    drain_out(_WINDOW - 1, 1)


def kernel(x, table, pos_encoding):
    out5 = _emb_pe_kernel(x.T, table, pos_encoding)
    return out5.transpose(2, 4, 0, 1, 3).reshape(_BATCH, _WINDOW, _EMBED)


# speedup vs baseline: 1.4809x; 1.4809x over previous
import functools
import math

import jax
import jax.numpy as jnp
from jax import lax
from jax.experimental import pallas as pl
from jax.experimental.pallas import tpu as pltpu
from jax.experimental.pallas import tpu_sc as plsc

_BATCH = 4096
_WINDOW = 200
_EMBED = 64
_NC, _NS = 2, 16
_NW = _NC * _NS
_BSL = _BATCH // _NW
_SCALE = math.sqrt(_EMBED)
_L = 16
_ET = _EMBED // 8
_WB = 2
_NBLK = _WINDOW // _WB
_PITCH = 133

_mesh = plsc.VectorSubcoreMesh(core_axis_name="c", subcore_axis_name="s")


@functools.partial(
    pl.kernel,
    out_type=jax.ShapeDtypeStruct((_WINDOW, _ET, _NW, 8, _BSL), jnp.float32),
    mesh=_mesh,
    scratch_types=[
        pltpu.VMEM((_WINDOW, _BSL), jnp.int32),
        pltpu.VMEM((_WB, _BSL, _EMBED), jnp.float32),
        pltpu.VMEM((_WB, _BSL, _EMBED), jnp.float32),
        pltpu.VMEM((_WB, _EMBED, _PITCH), jnp.float32),
        pltpu.VMEM((_WB, _EMBED, _PITCH), jnp.float32),
        pltpu.VMEM((_WINDOW, _EMBED), jnp.float32),
        pltpu.SemaphoreType.DMA,
        pltpu.SemaphoreType.DMA,
        pltpu.SemaphoreType.DMA,
        pltpu.SemaphoreType.DMA,
    ],
    compiler_params=pltpu.CompilerParams(
        use_tc_tiling_on_sc=False, needs_layout_passes=False),
)
def _emb_pe_kernel(xt_hbm, table_hbm, pos_hbm, out_hbm,
                   idx_v, gbuf0, gbuf1, obuf0, obuf1, pos_v,
                   sg0, sg1, so0, so1):
    wid = lax.axis_index("s") * _NC + lax.axis_index("c")
    b0 = wid * _BSL
    gbufs, obufs = (gbuf0, gbuf1), (obuf0, obuf1)
    sgs, sos = (sg0, sg1), (so0, so1)

    pltpu.sync_copy(pos_hbm, pos_v)
    pltpu.sync_copy(xt_hbm.at[:, pl.ds(b0, _BSL)], idx_v)

    iota = jnp.arange(_L, dtype=jnp.int32)
    i_es = [s * _L + iota for s in range(_EMBED // _L)]

    def fire_gathers(g, b):
        for j in range(_WB):
            pltpu.async_copy(
                table_hbm.at[idx_v.at[g * _WB + j]], gbufs[b].at[j], sgs[b])

    def wait_gathers(g, b):
        for j in range(_WB):
            pltpu.make_async_copy(
                table_hbm.at[idx_v.at[g * _WB + j]], gbufs[b].at[j],
                sgs[b]).wait()

    def fire_out(g, b):
        for j in range(_WB):
            for e in range(_ET):
                pltpu.async_copy(
                    obufs[b].at[j, pl.ds(e * 8, 8), pl.ds(0, _BSL)],
                    out_hbm.at[g * _WB + j, e, wid], sos[b])

    def drain_out(g, b):
        for j in range(_WB):
            for e in range(_ET):
                pltpu.make_async_copy(
                    obufs[b].at[j, pl.ds(e * 8, 8), pl.ds(0, _BSL)],
                    out_hbm.at[g * _WB + j, e, wid], sos[b]).wait()

    def compute(g, b):
        for j in range(_WB):
            gbuf = gbufs[b].at[j]
            obuf = obufs[b].at[j]
            w = g * _WB + j
            ps = [pos_v[w, pl.ds(s * _L, _L)] for s in range(_EMBED // _L)]

            @pl.loop(0, _BSL, unroll=4)
            def _col(c):
                i2 = jnp.full((_L,), c, jnp.int32)
                for s in range(_EMBED // _L):
                    v = gbuf[c, pl.ds(s * _L, _L)] * _SCALE + ps[s]
                    plsc.store_scatter(obuf, (i_es[s], i2), v)

    fire_gathers(0, 0)

    @pl.loop(0, _NBLK, step=2)
    def _gloop(gb):
        for b in range(2):
            g = gb + b

            @pl.when(g + 1 < _NBLK)
            def _():
                fire_gathers(g + 1, 1 - b)

            wait_gathers(g, b)

            @pl.when(g >= 2)
            def _():
                drain_out(g - 2, b)

            compute(g, b)
            fire_out(g, b)

    drain_out(_NBLK - 2, 0)
    drain_out(_NBLK - 1, 1)


def kernel(x, table, pos_encoding):
    out5 = _emb_pe_kernel(x.T, table, pos_encoding)
    return out5.transpose(2, 4, 0, 1, 3).reshape(_BATCH, _WINDOW, _EMBED)


# speedup vs baseline: 1.4812x; 1.0001x over previous
import functools
import math

import jax
import jax.numpy as jnp
from jax import lax
from jax.experimental import pallas as pl
from jax.experimental.pallas import tpu as pltpu
from jax.experimental.pallas import tpu_sc as plsc

_BATCH = 4096
_WINDOW = 200
_EMBED = 64
_NC, _NS = 2, 16
_NW = _NC * _NS
_BSL = _BATCH // _NW
_SCALE = math.sqrt(_EMBED)
_L = 16
_ET = _EMBED // 8
_WB = 2
_NBLK = _WINDOW // _WB
_PITCH = 133

_mesh = plsc.VectorSubcoreMesh(core_axis_name="c", subcore_axis_name="s")


@functools.partial(
    pl.kernel,
    out_type=jax.ShapeDtypeStruct((_WINDOW, _ET, _NW, 8, _BSL), jnp.float32),
    mesh=_mesh,
    scratch_types=[
        pltpu.VMEM((_WINDOW, _BSL), jnp.int32),
        pltpu.VMEM((_WB, _BSL, _EMBED), jnp.float32),
        pltpu.VMEM((_WB, _BSL, _EMBED), jnp.float32),
        pltpu.VMEM((_WB, _ET, 8, _PITCH), jnp.float32),
        pltpu.VMEM((_WB, _ET, 8, _PITCH), jnp.float32),
        pltpu.VMEM((_WINDOW, _EMBED), jnp.float32),
        pltpu.SemaphoreType.DMA,
        pltpu.SemaphoreType.DMA,
        pltpu.SemaphoreType.DMA,
        pltpu.SemaphoreType.DMA,
    ],
    compiler_params=pltpu.CompilerParams(
        use_tc_tiling_on_sc=False, needs_layout_passes=False),
)
def _emb_pe_kernel(xt_hbm, table_hbm, pos_hbm, out_hbm,
                   idx_v, gbuf0, gbuf1, obuf0, obuf1, pos_v,
                   sg0, sg1, so0, so1):
    wid = lax.axis_index("s") * _NC + lax.axis_index("c")
    b0 = wid * _BSL
    gbufs, obufs = (gbuf0, gbuf1), (obuf0, obuf1)
    sgs, sos = (sg0, sg1), (so0, so1)

    pltpu.sync_copy(pos_hbm, pos_v)
    pltpu.sync_copy(xt_hbm.at[:, pl.ds(b0, _BSL)], idx_v)

    iota = jnp.arange(_L, dtype=jnp.int32)
    i0s = [(s * _L + iota) >> 3 for s in range(_EMBED // _L)]
    i1s = [(s * _L + iota) & 7 for s in range(_EMBED // _L)]

    def fire_gathers(g, b):
        for j in range(_WB):
            pltpu.async_copy(
                table_hbm.at[idx_v.at[g * _WB + j]], gbufs[b].at[j], sgs[b])

    def wait_gathers(g, b):
        for j in range(_WB):
            pltpu.make_async_copy(
                table_hbm.at[idx_v.at[g * _WB + j]], gbufs[b].at[j],
                sgs[b]).wait()

    def fire_out(g, b):
        for j in range(_WB):
            pltpu.async_copy(
                obufs[b].at[j, :, :, pl.ds(0, _BSL)],
                out_hbm.at[g * _WB + j, :, wid], sos[b])

    def drain_out(g, b):
        for j in range(_WB):
            pltpu.make_async_copy(
                obufs[b].at[j, :, :, pl.ds(0, _BSL)],
                out_hbm.at[g * _WB + j, :, wid], sos[b]).wait()

    def compute(g, b):
        for j in range(_WB):
            gbuf = gbufs[b].at[j]
            obuf = obufs[b].at[j]
            w = g * _WB + j
            ps = [pos_v[w, pl.ds(s * _L, _L)] for s in range(_EMBED // _L)]

            @pl.loop(0, _BSL, unroll=4)
            def _col(c):
                i2 = jnp.full((_L,), c, jnp.int32)
                for s in range(_EMBED // _L):
                    v = gbuf[c, pl.ds(s * _L, _L)] * _SCALE + ps[s]
                    plsc.store_scatter(obuf, (i0s[s], i1s[s], i2), v)

    fire_gathers(0, 0)

    @pl.loop(0, _NBLK, step=2)
    def _gloop(gb):
        for b in range(2):
            g = gb + b

            @pl.when(g + 1 < _NBLK)
            def _():
                fire_gathers(g + 1, 1 - b)

            wait_gathers(g, b)

            @pl.when(g >= 2)
            def _():
                drain_out(g - 2, b)

            compute(g, b)
            fire_out(g, b)

    drain_out(_NBLK - 2, 0)
    drain_out(_NBLK - 1, 1)


def kernel(x, table, pos_encoding):
    out5 = _emb_pe_kernel(x.T, table, pos_encoding)
    return out5.transpose(2, 4, 0, 1, 3).reshape(_BATCH, _WINDOW, _EMBED)


# speedup vs baseline: 1.4840x; 1.0019x over previous
import functools
import math

import jax
import jax.numpy as jnp
from jax import lax
from jax.experimental import pallas as pl
from jax.experimental.pallas import tpu as pltpu
from jax.experimental.pallas import tpu_sc as plsc

_BATCH = 4096
_WINDOW = 200
_EMBED = 64
_NC, _NS = 2, 16
_NW = _NC * _NS
_BSL = _BATCH // _NW
_SCALE = math.sqrt(_EMBED)
_L = 16
_ET = _EMBED // 8
_WB = 2
_NBLK = _WINDOW // _WB
_PITCH = 133

_mesh = plsc.VectorSubcoreMesh(core_axis_name="c", subcore_axis_name="s")


@functools.partial(
    pl.kernel,
    out_type=jax.ShapeDtypeStruct((_WINDOW, _ET, _NW, 8, _BSL), jnp.float32),
    mesh=_mesh,
    scratch_types=[
        pltpu.VMEM((_WINDOW * _BSL,), jnp.int32),
        pltpu.VMEM((_WB * _BSL, _EMBED), jnp.float32),
        pltpu.VMEM((_WB * _BSL, _EMBED), jnp.float32),
        pltpu.VMEM((_WB, _ET, 8, _PITCH), jnp.float32),
        pltpu.VMEM((_WB, _ET, 8, _PITCH), jnp.float32),
        pltpu.VMEM((_WINDOW, _EMBED), jnp.float32),
        pltpu.SemaphoreType.DMA,
        pltpu.SemaphoreType.DMA,
        pltpu.SemaphoreType.DMA,
        pltpu.SemaphoreType.DMA,
    ],
    compiler_params=pltpu.CompilerParams(
        use_tc_tiling_on_sc=False, needs_layout_passes=False),
)
def _emb_pe_kernel(xt_hbm, table_hbm, pos_hbm, out_hbm,
                   idx_v, gbuf0, gbuf1, obuf0, obuf1, pos_v,
                   sg0, sg1, so0, so1):
    wid = lax.axis_index("s") * _NC + lax.axis_index("c")
    b0 = wid * _BSL
    gbufs, obufs = (gbuf0, gbuf1), (obuf0, obuf1)
    sgs, sos = (sg0, sg1), (so0, so1)

    pltpu.sync_copy(pos_hbm, pos_v)

    @pl.loop(0, _WINDOW)
    def _stage_idx(w):
        pltpu.async_copy(xt_hbm.at[w, pl.ds(b0, _BSL)],
                         idx_v.at[pl.ds(w * _BSL, _BSL)], so0)

    @pl.loop(0, _WINDOW)
    def _drain_idx(w):
        pltpu.make_async_copy(xt_hbm.at[w, pl.ds(b0, _BSL)],
                              idx_v.at[pl.ds(w * _BSL, _BSL)], so0).wait()

    iota = jnp.arange(_L, dtype=jnp.int32)
    i0s = [(s * _L + iota) >> 3 for s in range(_EMBED // _L)]
    i1s = [(s * _L + iota) & 7 for s in range(_EMBED // _L)]

    def fire_gathers(g, b):
        pltpu.async_copy(
            table_hbm.at[idx_v.at[pl.ds(g * _WB * _BSL, _WB * _BSL)]],
            gbufs[b], sgs[b])

    def wait_gathers(g, b):
        pltpu.make_async_copy(
            table_hbm.at[idx_v.at[pl.ds(g * _WB * _BSL, _WB * _BSL)]],
            gbufs[b], sgs[b]).wait()

    def fire_out(g, b):
        for j in range(_WB):
            pltpu.async_copy(
                obufs[b].at[j, :, :, pl.ds(0, _BSL)],
                out_hbm.at[g * _WB + j, :, wid], sos[b])

    def drain_out(g, b):
        for j in range(_WB):
            pltpu.make_async_copy(
                obufs[b].at[j, :, :, pl.ds(0, _BSL)],
                out_hbm.at[g * _WB + j, :, wid], sos[b]).wait()

    def compute(g, b):
        for j in range(_WB):
            gbuf = gbufs[b]
            obuf = obufs[b].at[j]
            w = g * _WB + j
            ps = [pos_v[w, pl.ds(s * _L, _L)] for s in range(_EMBED // _L)]

            @pl.loop(0, _BSL, unroll=4)
            def _col(c):
                i2 = jnp.full((_L,), c, jnp.int32)
                for s in range(_EMBED // _L):
                    v = gbuf[j * _BSL + c, pl.ds(s * _L, _L)] * _SCALE + ps[s]
                    plsc.store_scatter(obuf, (i0s[s], i1s[s], i2), v)

    fire_gathers(0, 0)

    @pl.loop(0, _NBLK, step=2)
    def _gloop(gb):
        for b in range(2):
            g = gb + b

            @pl.when(g + 1 < _NBLK)
            def _():
                fire_gathers(g + 1, 1 - b)

            wait_gathers(g, b)

            @pl.when(g >= 2)
            def _():
                drain_out(g - 2, b)

            compute(g, b)
            fire_out(g, b)

    drain_out(_NBLK - 2, 0)
    drain_out(_NBLK - 1, 1)


def kernel(x, table, pos_encoding):
    out5 = _emb_pe_kernel(x.T, table, pos_encoding)
    return out5.transpose(2, 4, 0, 1, 3).reshape(_BATCH, _WINDOW, _EMBED)
